# baseline (speedup 1.0000x reference)
import functools

import jax
import jax.numpy as jnp
from jax import lax
from jax.experimental import pallas as pl
from jax.experimental.pallas import tpu as pltpu
from jax.experimental.pallas import tpu_sc as plsc

_N_EMBEDS = 1024
_EMBED_DIM = 64
_BETA = 0.25

_B = 32
_T = 1024
_LANES = 128
_SUB = 8
_NC = 2
_NS = 16
_NW = _NC * _NS


def _tc_body(xt_ref, embt_ref, idx_ref, loss_ref):
    pid = pl.program_id(0)
    nblocks = pl.num_programs(0)

    xb = xt_ref[0]
    embt = embt_ref[...]
    sim = jnp.dot(embt, xb, preferred_element_type=jnp.float32)
    x2 = jnp.sum(xb * xb, axis=0, keepdims=True)
    e2 = jnp.sum(embt * embt, axis=1, keepdims=True)
    dists = x2 + e2 - 2.0 * sim

    minv = jnp.min(dists, axis=0, keepdims=True)
    rows = lax.broadcasted_iota(jnp.int32, dists.shape, 0)
    idx = jnp.min(jnp.where(dists == minv, rows, _N_EMBEDS), axis=0)
    idx_ref[...] = idx.reshape(_T // _LANES, _LANES)

    @pl.when(pid == 0)
    def _init():
        loss_ref[0, 0] = 0.0

    loss_ref[0, 0] += jnp.sum(minv)

    @pl.when(pid == nblocks - 1)
    def _finish():
        total = jnp.float32(_B * _T * _EMBED_DIM)
        loss_ref[0, 0] = loss_ref[0, 0] * ((1.0 + _BETA) / total)


def _tc_stage(xt, embt):
    idx_rows_blk = _T // _LANES
    return pl.pallas_call(
        _tc_body,
        grid=(_B,),
        in_specs=[
            pl.BlockSpec((1, _EMBED_DIM, _T), lambda i: (i, 0, 0)),
            pl.BlockSpec((_N_EMBEDS, _EMBED_DIM), lambda i: (0, 0)),
        ],
        out_specs=[
            pl.BlockSpec((idx_rows_blk, _LANES), lambda i: (i, 0)),
            pl.BlockSpec(memory_space=pltpu.SMEM, block_shape=(1, 1),
                         index_map=lambda i: (0, 0)),
        ],
        out_shape=[
            jax.ShapeDtypeStruct((_B * idx_rows_blk, _LANES), jnp.int32),
            jax.ShapeDtypeStruct((1, 1), jnp.float32),
        ],
    )(xt, embt)


def _make_sc_gather():
    tchunks = _T // _LANES
    _PITCH = _EMBED_DIM + 1
    mesh = plsc.VectorSubcoreMesh(core_axis_name="c", subcore_axis_name="s")

    @functools.partial(
        pl.kernel,
        mesh=mesh,
        out_type=jax.ShapeDtypeStruct(
            (_B, _SUB, tchunks, _SUB, _LANES), jnp.float32),
        scratch_types=[
            pltpu.VMEM((_N_EMBEDS, _PITCH), jnp.float32),
            pltpu.VMEM((tchunks, _LANES), jnp.int32),
            pltpu.VMEM((_EMBED_DIM, _LANES), jnp.float32),
            pltpu.VMEM((_EMBED_DIM, _LANES), jnp.float32),
            pltpu.SemaphoreType.DMA,
            pltpu.SemaphoreType.DMA,
            pltpu.SemaphoreType.DMA,
        ],
        compiler_params=pltpu.CompilerParams(use_tc_tiling_on_sc=False,
                                             needs_layout_passes=False),
    )
    def _sc_gather(table_hbm, idx_hbm, out_hbm, table_v, idx_v, buf0, buf1,
                   tsem, wsem0, wsem1):
        b = lax.axis_index("s") * _NC + lax.axis_index("c")
        pltpu.sync_copy(idx_hbm.at[pl.ds(b * tchunks, tchunks)], idx_v)
        pltpu.sync_copy(table_hbm, table_v.at[:, pl.ds(0, _EMBED_DIM)])

        cols = [jnp.full((16,), e, jnp.int32) for e in range(_EMBED_DIM)]

        def assemble(tc, buf):
            for g in range(_LANES // 16):
                rows = idx_v[tc, pl.ds(g * 16, 16)]
                for e in range(_EMBED_DIM):
                    buf[e, pl.ds(g * 16, 16)] = plsc.load_gather(
                        table_v, [rows, cols[e]])

        def fire(tc, buf, wsem):
            for ts in range(_SUB):
                pltpu.async_copy(buf.at[pl.ds(ts * _SUB, _SUB)],
                                 out_hbm.at[b, ts, tc], wsem)

        def drain(tc, buf, wsem):
            for ts in range(_SUB):
                pltpu.make_async_copy(buf.at[pl.ds(ts * _SUB, _SUB)],
                                      out_hbm.at[b, ts, tc], wsem).wait()

        def body(i, carry):
            tc0 = 2 * i
            tc1 = tc0 + 1

            @pl.when(i > 0)
            def _():
                drain(tc0 - 2, buf0, wsem0)

            assemble(tc0, buf0)
            fire(tc0, buf0, wsem0)

            @pl.when(i > 0)
            def _():
                drain(tc1 - 2, buf1, wsem1)

            assemble(tc1, buf1)
            fire(tc1, buf1, wsem1)
            return carry

        lax.fori_loop(0, tchunks // 2, body, 0)
        drain(tchunks - 2, buf0, wsem0)
        drain(tchunks - 1, buf1, wsem1)

    return _sc_gather


def kernel(x, embeddings):
    xt = jnp.swapaxes(x, 1, 2)
    embt = embeddings.T
    idx, loss = _tc_stage(xt, embt)

    out5 = _make_sc_gather()(embt, idx)
    qtised = out5.transpose(0, 2, 4, 1, 3).reshape(_B, _T, _EMBED_DIM)
    return (qtised, loss.reshape(()))


# speedup vs baseline: 1.2731x; 1.2731x over previous
import functools

import jax
import jax.numpy as jnp
from jax import lax
from jax.experimental import pallas as pl
from jax.experimental.pallas import tpu as pltpu
from jax.experimental.pallas import tpu_sc as plsc

_N_EMBEDS = 1024
_EMBED_DIM = 64
_BETA = 0.25

_B = 32
_T = 1024
_LANES = 128
_SUB = 8
_NC = 2
_NS = 16
_NW = _NC * _NS


def _tc_body(xt_ref, embt_ref, idx_ref, loss_ref):
    pid = pl.program_id(0)
    nblocks = pl.num_programs(0)

    xb = xt_ref[0]
    embt = embt_ref[...]
    sim = jnp.dot(embt, xb, preferred_element_type=jnp.float32)
    x2 = jnp.sum(xb * xb, axis=0, keepdims=True)
    e2 = jnp.sum(embt * embt, axis=1, keepdims=True)
    dists = x2 + e2 - 2.0 * sim

    minv = jnp.min(dists, axis=0, keepdims=True)
    rows = lax.broadcasted_iota(jnp.int32, dists.shape, 0)
    idx = jnp.min(jnp.where(dists == minv, rows, _N_EMBEDS), axis=0)
    idx_ref[...] = idx.reshape(_T // _LANES, _LANES)

    @pl.when(pid == 0)
    def _init():
        loss_ref[0, 0] = 0.0

    loss_ref[0, 0] += jnp.sum(minv)

    @pl.when(pid == nblocks - 1)
    def _finish():
        total = jnp.float32(_B * _T * _EMBED_DIM)
        loss_ref[0, 0] = loss_ref[0, 0] * ((1.0 + _BETA) / total)


def _tc_stage(xt, embt):
    idx_rows_blk = _T // _LANES
    return pl.pallas_call(
        _tc_body,
        grid=(_B,),
        in_specs=[
            pl.BlockSpec((1, _EMBED_DIM, _T), lambda i: (i, 0, 0)),
            pl.BlockSpec((_N_EMBEDS, _EMBED_DIM), lambda i: (0, 0)),
        ],
        out_specs=[
            pl.BlockSpec((idx_rows_blk, _LANES), lambda i: (i, 0)),
            pl.BlockSpec(memory_space=pltpu.SMEM, block_shape=(1, 1),
                         index_map=lambda i: (0, 0)),
        ],
        out_shape=[
            jax.ShapeDtypeStruct((_B * idx_rows_blk, _LANES), jnp.int32),
            jax.ShapeDtypeStruct((1, 1), jnp.float32),
        ],
    )(xt, embt)


def _make_sc_gather():
    tchunks = _T // _LANES
    _PITCH = _EMBED_DIM + 1
    mesh = plsc.VectorSubcoreMesh(core_axis_name="c", subcore_axis_name="s")

    @functools.partial(
        pl.kernel,
        mesh=mesh,
        out_type=jax.ShapeDtypeStruct(
            (_B, _SUB, tchunks, _SUB, _LANES), jnp.float32),
        scratch_types=[
            pltpu.VMEM((_N_EMBEDS, _PITCH), jnp.float32),
            pltpu.VMEM((tchunks, _LANES), jnp.int32),
            pltpu.VMEM((_EMBED_DIM, _LANES), jnp.float32),
            pltpu.VMEM((_EMBED_DIM, _LANES), jnp.float32),
            pltpu.SemaphoreType.DMA,
            pltpu.SemaphoreType.DMA,
            pltpu.SemaphoreType.DMA,
        ],
        compiler_params=pltpu.CompilerParams(use_tc_tiling_on_sc=False,
                                             needs_layout_passes=False),
    )
    def _sc_gather(table_hbm, idx_hbm, out_hbm, table_v, idx_v, buf0, buf1,
                   tsem, wsem0, wsem1):
        b = lax.axis_index("s") * _NC + lax.axis_index("c")
        pltpu.sync_copy(idx_hbm.at[pl.ds(b * tchunks, tchunks)], idx_v)
        pltpu.sync_copy(table_hbm, table_v.at[:, pl.ds(0, _EMBED_DIM)])

        def assemble(tc, buf):
            for g in range(_LANES // 16):
                rows = idx_v[tc, pl.ds(g * 16, 16)]

                @plsc.parallel_loop(0, _EMBED_DIM, unroll=8)
                def _(e):
                    col = jnp.full((16,), 0, jnp.int32) + e
                    buf[e, pl.ds(g * 16, 16)] = plsc.load_gather(
                        table_v, [rows, col])

        def fire(tc, buf, wsem):
            for ts in range(_SUB):
                pltpu.async_copy(buf.at[pl.ds(ts * _SUB, _SUB)],
                                 out_hbm.at[b, ts, tc], wsem)

        def drain(tc, buf, wsem):
            for ts in range(_SUB):
                pltpu.make_async_copy(buf.at[pl.ds(ts * _SUB, _SUB)],
                                      out_hbm.at[b, ts, tc], wsem).wait()

        def body(i, carry):
            tc0 = 2 * i
            tc1 = tc0 + 1

            @pl.when(i > 0)
            def _():
                drain(tc0 - 2, buf0, wsem0)

            assemble(tc0, buf0)
            fire(tc0, buf0, wsem0)

            @pl.when(i > 0)
            def _():
                drain(tc1 - 2, buf1, wsem1)

            assemble(tc1, buf1)
            fire(tc1, buf1, wsem1)
            return carry

        lax.fori_loop(0, tchunks // 2, body, 0)
        drain(tchunks - 2, buf0, wsem0)
        drain(tchunks - 1, buf1, wsem1)

    return _sc_gather


def kernel(x, embeddings):
    xt = jnp.swapaxes(x, 1, 2)
    embt = embeddings.T
    idx, loss = _tc_stage(xt, embt)

    out5 = _make_sc_gather()(embt, idx)
    qtised = out5.transpose(0, 2, 4, 1, 3).reshape(_B, _T, _EMBED_DIM)
    return (qtised, loss.reshape(()))


# speedup vs baseline: 1.3302x; 1.0449x over previous
import functools

import jax
import jax.numpy as jnp
from jax import lax
from jax.experimental import pallas as pl
from jax.experimental.pallas import tpu as pltpu
from jax.experimental.pallas import tpu_sc as plsc

_N_EMBEDS = 1024
_EMBED_DIM = 64
_BETA = 0.25

_B = 32
_T = 1024
_LANES = 128
_SUB = 8
_NC = 2
_NS = 16
_NW = _NC * _NS


def _tc_body(xt_ref, embt_ref, idx_ref, loss_ref):
    pid = pl.program_id(0)
    nblocks = pl.num_programs(0)

    xb = xt_ref[0]
    embt = embt_ref[...]
    sim = jnp.dot(embt, xb, preferred_element_type=jnp.float32)
    x2 = jnp.sum(xb * xb, axis=0, keepdims=True)
    e2 = jnp.sum(embt * embt, axis=1, keepdims=True)
    dists = x2 + e2 - 2.0 * sim

    minv = jnp.min(dists, axis=0, keepdims=True)
    idx = jnp.argmin(dists, axis=0).astype(jnp.int32)
    idx_ref[...] = idx.reshape(_T // _LANES, _LANES)

    @pl.when(pid == 0)
    def _init():
        loss_ref[0, 0] = 0.0

    loss_ref[0, 0] += jnp.sum(minv)

    @pl.when(pid == nblocks - 1)
    def _finish():
        total = jnp.float32(_B * _T * _EMBED_DIM)
        loss_ref[0, 0] = loss_ref[0, 0] * ((1.0 + _BETA) / total)


def _tc_stage(xt, embt):
    idx_rows_blk = _T // _LANES
    return pl.pallas_call(
        _tc_body,
        grid=(_B,),
        in_specs=[
            pl.BlockSpec((1, _EMBED_DIM, _T), lambda i: (i, 0, 0)),
            pl.BlockSpec((_N_EMBEDS, _EMBED_DIM), lambda i: (0, 0)),
        ],
        out_specs=[
            pl.BlockSpec((idx_rows_blk, _LANES), lambda i: (i, 0)),
            pl.BlockSpec(memory_space=pltpu.SMEM, block_shape=(1, 1),
                         index_map=lambda i: (0, 0)),
        ],
        out_shape=[
            jax.ShapeDtypeStruct((_B * idx_rows_blk, _LANES), jnp.int32),
            jax.ShapeDtypeStruct((1, 1), jnp.float32),
        ],
    )(xt, embt)


def _make_sc_gather():
    tchunks = _T // _LANES
    _PITCH = _EMBED_DIM + 1
    mesh = plsc.VectorSubcoreMesh(core_axis_name="c", subcore_axis_name="s")

    @functools.partial(
        pl.kernel,
        mesh=mesh,
        out_type=jax.ShapeDtypeStruct(
            (_B, _SUB, tchunks, _SUB, _LANES), jnp.float32),
        scratch_types=[
            pltpu.VMEM((_N_EMBEDS, _PITCH), jnp.float32),
            pltpu.VMEM((tchunks, _LANES), jnp.int32),
            pltpu.VMEM((_EMBED_DIM, _LANES), jnp.float32),
            pltpu.VMEM((_EMBED_DIM, _LANES), jnp.float32),
            pltpu.SemaphoreType.DMA,
            pltpu.SemaphoreType.DMA,
            pltpu.SemaphoreType.DMA,
        ],
        compiler_params=pltpu.CompilerParams(use_tc_tiling_on_sc=False,
                                             needs_layout_passes=False),
    )
    def _sc_gather(table_hbm, idx_hbm, out_hbm, table_v, idx_v, buf0, buf1,
                   tsem, wsem0, wsem1):
        b = lax.axis_index("s") * _NC + lax.axis_index("c")
        c1 = pltpu.async_copy(idx_hbm.at[pl.ds(b * tchunks, tchunks)], idx_v,
                              tsem)
        c2 = pltpu.async_copy(table_hbm, table_v, tsem)
        c1.wait()
        c2.wait()

        def assemble(tc, buf):
            for g in range(_LANES // 16):
                rows = idx_v[tc, pl.ds(g * 16, 16)]

                @plsc.parallel_loop(0, _EMBED_DIM, unroll=16)
                def _(e):
                    col = jnp.full((16,), 0, jnp.int32) + e
                    buf[e, pl.ds(g * 16, 16)] = plsc.load_gather(
                        table_v, [rows, col])

        def fire(tc, buf, wsem):
            for ts in range(_SUB):
                pltpu.async_copy(buf.at[pl.ds(ts * _SUB, _SUB)],
                                 out_hbm.at[b, ts, tc], wsem)

        def drain(tc, buf, wsem):
            for ts in range(_SUB):
                pltpu.make_async_copy(buf.at[pl.ds(ts * _SUB, _SUB)],
                                      out_hbm.at[b, ts, tc], wsem).wait()

        def body(i, carry):
            tc0 = 2 * i
            tc1 = tc0 + 1

            @pl.when(i > 0)
            def _():
                drain(tc0 - 2, buf0, wsem0)

            assemble(tc0, buf0)
            fire(tc0, buf0, wsem0)

            @pl.when(i > 0)
            def _():
                drain(tc1 - 2, buf1, wsem1)

            assemble(tc1, buf1)
            fire(tc1, buf1, wsem1)
            return carry

        lax.fori_loop(0, tchunks // 2, body, 0)
        drain(tchunks - 2, buf0, wsem0)
        drain(tchunks - 1, buf1, wsem1)

    return _sc_gather


def kernel(x, embeddings):
    xt = jnp.swapaxes(x, 1, 2)
    embt = embeddings.T
    idx, loss = _tc_stage(xt, embt)

    embt_pad = jnp.pad(embt, ((0, 0), (0, 1)))
    out5 = _make_sc_gather()(embt_pad, idx)
    qtised = out5.transpose(0, 2, 4, 1, 3).reshape(_B, _T, _EMBED_DIM)
    return (qtised, loss.reshape(()))


# speedup vs baseline: 1.3786x; 1.0364x over previous
import functools

import jax
import jax.numpy as jnp
from jax import lax
from jax.experimental import pallas as pl
from jax.experimental.pallas import tpu as pltpu
from jax.experimental.pallas import tpu_sc as plsc

_N_EMBEDS = 1024
_EMBED_DIM = 64
_BETA = 0.25

_B = 32
_T = 1024
_LANES = 128
_SUB = 8
_NC = 2
_NS = 16
_NW = _NC * _NS
_BPB = 2


def _tc_body(xt_ref, embt_ref, idx_ref, loss_ref):
    pid = pl.program_id(0)
    nblocks = pl.num_programs(0)

    embt = embt_ref[...]
    e2 = jnp.sum(embt * embt, axis=1, keepdims=True)
    for j in range(_BPB):
        xb = xt_ref[j]
        sim = jnp.dot(embt, xb, preferred_element_type=jnp.float32)
        x2 = jnp.sum(xb * xb, axis=0, keepdims=True)
        dists = x2 + e2 - 2.0 * sim

        minv = jnp.min(dists, axis=0, keepdims=True)
        idx = jnp.argmin(dists, axis=0).astype(jnp.int32)
        idx_ref[pl.ds(j * (_T // _LANES), _T // _LANES)] = idx.reshape(
            _T // _LANES, _LANES)
        if j == 0:
            msum = jnp.sum(minv)
        else:
            msum = msum + jnp.sum(minv)

    @pl.when(pid == 0)
    def _init():
        loss_ref[0, 0] = 0.0

    loss_ref[0, 0] += msum

    @pl.when(pid == nblocks - 1)
    def _finish():
        total = jnp.float32(_B * _T * _EMBED_DIM)
        loss_ref[0, 0] = loss_ref[0, 0] * ((1.0 + _BETA) / total)


def _tc_stage(xt, embt):
    idx_rows_blk = _BPB * (_T // _LANES)
    return pl.pallas_call(
        _tc_body,
        grid=(_B // _BPB,),
        in_specs=[
            pl.BlockSpec((_BPB, _EMBED_DIM, _T), lambda i: (i, 0, 0)),
            pl.BlockSpec((_N_EMBEDS, _EMBED_DIM), lambda i: (0, 0)),
        ],
        out_specs=[
            pl.BlockSpec((idx_rows_blk, _LANES), lambda i: (i, 0)),
            pl.BlockSpec(memory_space=pltpu.SMEM, block_shape=(1, 1),
                         index_map=lambda i: (0, 0)),
        ],
        out_shape=[
            jax.ShapeDtypeStruct((_B * (_T // _LANES), _LANES), jnp.int32),
            jax.ShapeDtypeStruct((1, 1), jnp.float32),
        ],
    )(xt, embt)


def _make_sc_gather():
    tchunks = _T // _LANES
    _PITCH = _EMBED_DIM + 1
    mesh = plsc.VectorSubcoreMesh(core_axis_name="c", subcore_axis_name="s")

    @functools.partial(
        pl.kernel,
        mesh=mesh,
        out_type=jax.ShapeDtypeStruct(
            (_B, _SUB, tchunks, _SUB, _LANES), jnp.float32),
        scratch_types=[
            pltpu.VMEM((_N_EMBEDS, _PITCH), jnp.float32),
            pltpu.VMEM((tchunks, _LANES), jnp.int32),
            pltpu.VMEM((_EMBED_DIM, _LANES), jnp.float32),
            pltpu.VMEM((_EMBED_DIM, _LANES), jnp.float32),
            pltpu.SemaphoreType.DMA,
            pltpu.SemaphoreType.DMA,
            pltpu.SemaphoreType.DMA,
        ],
        compiler_params=pltpu.CompilerParams(use_tc_tiling_on_sc=False,
                                             needs_layout_passes=False),
    )
    def _sc_gather(table_hbm, idx_hbm, out_hbm, table_v, idx_v, buf0, buf1,
                   tsem, wsem0, wsem1):
        b = lax.axis_index("s") * _NC + lax.axis_index("c")
        c1 = pltpu.async_copy(idx_hbm.at[pl.ds(b * tchunks, tchunks)], idx_v,
                              tsem)
        c2 = pltpu.async_copy(table_hbm, table_v, tsem)
        c1.wait()
        c2.wait()

        def assemble(tc, buf):
            for g in range(_LANES // 16):
                rows = idx_v[tc, pl.ds(g * 16, 16)]

                @plsc.parallel_loop(0, _EMBED_DIM, unroll=16)
                def _(e):
                    col = jnp.full((16,), 0, jnp.int32) + e
                    buf[e, pl.ds(g * 16, 16)] = plsc.load_gather(
                        table_v, [rows, col])

        def fire(tc, buf, wsem):
            for ts in range(_SUB):
                pltpu.async_copy(buf.at[pl.ds(ts * _SUB, _SUB)],
                                 out_hbm.at[b, ts, tc], wsem)

        def drain(tc, buf, wsem):
            for ts in range(_SUB):
                pltpu.make_async_copy(buf.at[pl.ds(ts * _SUB, _SUB)],
                                      out_hbm.at[b, ts, tc], wsem).wait()

        def body(i, carry):
            tc0 = 2 * i
            tc1 = tc0 + 1

            @pl.when(i > 0)
            def _():
                drain(tc0 - 2, buf0, wsem0)

            assemble(tc0, buf0)
            fire(tc0, buf0, wsem0)

            @pl.when(i > 0)
            def _():
                drain(tc1 - 2, buf1, wsem1)

            assemble(tc1, buf1)
            fire(tc1, buf1, wsem1)
            return carry

        lax.fori_loop(0, tchunks // 2, body, 0)
        drain(tchunks - 2, buf0, wsem0)
        drain(tchunks - 1, buf1, wsem1)

    return _sc_gather


def kernel(x, embeddings):
    xt = jnp.swapaxes(x, 1, 2)
    embt = embeddings.T
    idx, loss = _tc_stage(xt, embt)

    embt_pad = jnp.pad(embt, ((0, 0), (0, 1)))
    out5 = _make_sc_gather()(embt_pad, idx)
    qtised = out5.transpose(0, 2, 4, 1, 3).reshape(_B, _T, _EMBED_DIM)
    return (qtised, loss.reshape(()))


# speedup vs baseline: 1.4115x; 1.0238x over previous
import functools

import jax
import jax.numpy as jnp
from jax import lax
from jax.experimental import pallas as pl
from jax.experimental.pallas import tpu as pltpu
from jax.experimental.pallas import tpu_sc as plsc

_N_EMBEDS = 1024
_EMBED_DIM = 64
_BETA = 0.25

_B = 32
_T = 1024
_LANES = 128
_SUB = 8
_NC = 2
_NS = 16
_NW = _NC * _NS
_BPB = 4


def _tc_body(xt_ref, embt_ref, idx_ref, loss_ref):
    pid = pl.program_id(0)
    nblocks = pl.num_programs(0)

    embt = embt_ref[...]
    e2 = jnp.sum(embt * embt, axis=1, keepdims=True)
    for j in range(_BPB):
        xb = xt_ref[j]
        sim = jnp.dot(embt, xb, preferred_element_type=jnp.float32)
        x2 = jnp.sum(xb * xb, axis=0, keepdims=True)
        dists = x2 + e2 - 2.0 * sim

        minv = jnp.min(dists, axis=0, keepdims=True)
        idx = jnp.argmin(dists, axis=0).astype(jnp.int32)
        idx_ref[pl.ds(j * (_T // _LANES), _T // _LANES)] = idx.reshape(
            _T // _LANES, _LANES)
        if j == 0:
            msum = jnp.sum(minv)
        else:
            msum = msum + jnp.sum(minv)

    @pl.when(pid == 0)
    def _init():
        loss_ref[0, 0] = 0.0

    loss_ref[0, 0] += msum

    @pl.when(pid == nblocks - 1)
    def _finish():
        total = jnp.float32(_B * _T * _EMBED_DIM)
        loss_ref[0, 0] = loss_ref[0, 0] * ((1.0 + _BETA) / total)


def _tc_stage(xt, embt):
    idx_rows_blk = _BPB * (_T // _LANES)
    return pl.pallas_call(
        _tc_body,
        grid=(_B // _BPB,),
        in_specs=[
            pl.BlockSpec((_BPB, _EMBED_DIM, _T), lambda i: (i, 0, 0)),
            pl.BlockSpec((_N_EMBEDS, _EMBED_DIM), lambda i: (0, 0)),
        ],
        out_specs=[
            pl.BlockSpec((idx_rows_blk, _LANES), lambda i: (i, 0)),
            pl.BlockSpec(memory_space=pltpu.SMEM, block_shape=(1, 1),
                         index_map=lambda i: (0, 0)),
        ],
        out_shape=[
            jax.ShapeDtypeStruct((_B * (_T // _LANES), _LANES), jnp.int32),
            jax.ShapeDtypeStruct((1, 1), jnp.float32),
        ],
    )(xt, embt)


def _make_sc_gather():
    tchunks = _T // _LANES
    _PITCH = _EMBED_DIM + 1
    mesh = plsc.VectorSubcoreMesh(core_axis_name="c", subcore_axis_name="s")

    @functools.partial(
        pl.kernel,
        mesh=mesh,
        out_type=jax.ShapeDtypeStruct(
            (_B, _SUB, tchunks, _SUB, _LANES), jnp.float32),
        scratch_types=[
            pltpu.VMEM((_N_EMBEDS, _PITCH), jnp.float32),
            pltpu.VMEM((tchunks, _LANES), jnp.int32),
            pltpu.VMEM((_EMBED_DIM, _LANES), jnp.float32),
            pltpu.VMEM((_EMBED_DIM, _LANES), jnp.float32),
            pltpu.SemaphoreType.DMA,
            pltpu.SemaphoreType.DMA,
            pltpu.SemaphoreType.DMA,
        ],
        compiler_params=pltpu.CompilerParams(use_tc_tiling_on_sc=False,
                                             needs_layout_passes=False),
    )
    def _sc_gather(table_hbm, idx_hbm, out_hbm, table_v, idx_v, buf0, buf1,
                   tsem, wsem0, wsem1):
        b = lax.axis_index("s") * _NC + lax.axis_index("c")
        c1 = pltpu.async_copy(idx_hbm.at[pl.ds(b * tchunks, tchunks)], idx_v,
                              tsem)
        c2 = pltpu.async_copy(table_hbm, table_v, tsem)
        c1.wait()
        c2.wait()

        def assemble(tc, buf):
            for g in range(_LANES // 16):
                rows = idx_v[tc, pl.ds(g * 16, 16)]

                @plsc.parallel_loop(0, _EMBED_DIM, unroll=16)
                def _(e):
                    col = jnp.full((16,), 0, jnp.int32) + e
                    buf[e, pl.ds(g * 16, 16)] = plsc.load_gather(
                        table_v, [rows, col])

        def fire(tc, buf, wsem):
            for ts in range(_SUB):
                pltpu.async_copy(buf.at[pl.ds(ts * _SUB, _SUB)],
                                 out_hbm.at[b, ts, tc], wsem)

        def drain(tc, buf, wsem):
            for ts in range(_SUB):
                pltpu.make_async_copy(buf.at[pl.ds(ts * _SUB, _SUB)],
                                      out_hbm.at[b, ts, tc], wsem).wait()

        def body(i, carry):
            tc0 = 2 * i
            tc1 = tc0 + 1

            @pl.when(i > 0)
            def _():
                drain(tc0 - 2, buf0, wsem0)

            assemble(tc0, buf0)
            fire(tc0, buf0, wsem0)

            @pl.when(i > 0)
            def _():
                drain(tc1 - 2, buf1, wsem1)

            assemble(tc1, buf1)
            fire(tc1, buf1, wsem1)
            return carry

        lax.fori_loop(0, tchunks // 2, body, 0)
        drain(tchunks - 2, buf0, wsem0)
        drain(tchunks - 1, buf1, wsem1)

    return _sc_gather


def kernel(x, embeddings):
    xt = jnp.swapaxes(x, 1, 2)
    embt = embeddings.T
    idx, loss = _tc_stage(xt, embt)

    embt_pad = jnp.pad(embt, ((0, 0), (0, 1)))
    out5 = _make_sc_gather()(embt_pad, idx)
    qtised = out5.transpose(0, 2, 4, 1, 3).reshape(_B, _T, _EMBED_DIM)
    return (qtised, loss.reshape(()))


# speedup vs baseline: 1.4123x; 1.0006x over previous
import functools

import jax
import jax.numpy as jnp
from jax import lax
from jax.experimental import pallas as pl
from jax.experimental.pallas import tpu as pltpu
from jax.experimental.pallas import tpu_sc as plsc

_N_EMBEDS = 1024
_EMBED_DIM = 64
_BETA = 0.25

_B = 32
_T = 1024
_LANES = 128
_SUB = 8
_NC = 2
_NS = 16
_NW = _NC * _NS
_BPB = 8


def _tc_body(xt_ref, embt_ref, idx_ref, loss_ref):
    pid = pl.program_id(0)
    nblocks = pl.num_programs(0)

    embt = embt_ref[...]
    e2 = jnp.sum(embt * embt, axis=1, keepdims=True)
    for j in range(_BPB):
        xb = xt_ref[j]
        sim = jnp.dot(embt, xb, preferred_element_type=jnp.float32)
        x2 = jnp.sum(xb * xb, axis=0, keepdims=True)
        dists = x2 + e2 - 2.0 * sim

        minv = jnp.min(dists, axis=0, keepdims=True)
        idx = jnp.argmin(dists, axis=0).astype(jnp.int32)
        idx_ref[pl.ds(j * (_T // _LANES), _T // _LANES)] = idx.reshape(
            _T // _LANES, _LANES)
        if j == 0:
            msum = jnp.sum(minv)
        else:
            msum = msum + jnp.sum(minv)

    @pl.when(pid == 0)
    def _init():
        loss_ref[0, 0] = 0.0

    loss_ref[0, 0] += msum

    @pl.when(pid == nblocks - 1)
    def _finish():
        total = jnp.float32(_B * _T * _EMBED_DIM)
        loss_ref[0, 0] = loss_ref[0, 0] * ((1.0 + _BETA) / total)


def _tc_stage(xt, embt):
    idx_rows_blk = _BPB * (_T // _LANES)
    return pl.pallas_call(
        _tc_body,
        grid=(_B // _BPB,),
        in_specs=[
            pl.BlockSpec((_BPB, _EMBED_DIM, _T), lambda i: (i, 0, 0)),
            pl.BlockSpec((_N_EMBEDS, _EMBED_DIM), lambda i: (0, 0)),
        ],
        out_specs=[
            pl.BlockSpec((idx_rows_blk, _LANES), lambda i: (i, 0)),
            pl.BlockSpec(memory_space=pltpu.SMEM, block_shape=(1, 1),
                         index_map=lambda i: (0, 0)),
        ],
        out_shape=[
            jax.ShapeDtypeStruct((_B * (_T // _LANES), _LANES), jnp.int32),
            jax.ShapeDtypeStruct((1, 1), jnp.float32),
        ],
    )(xt, embt)


def _make_sc_gather():
    tchunks = _T // _LANES
    _PITCH = _EMBED_DIM + 1
    mesh = plsc.VectorSubcoreMesh(core_axis_name="c", subcore_axis_name="s")

    @functools.partial(
        pl.kernel,
        mesh=mesh,
        out_type=jax.ShapeDtypeStruct(
            (_B, _SUB, tchunks, _SUB, _LANES), jnp.float32),
        scratch_types=[
            pltpu.VMEM((_N_EMBEDS, _PITCH), jnp.float32),
            pltpu.VMEM((tchunks, _LANES), jnp.int32),
            pltpu.VMEM((_EMBED_DIM, _LANES), jnp.float32),
            pltpu.VMEM((_EMBED_DIM, _LANES), jnp.float32),
            pltpu.SemaphoreType.DMA,
            pltpu.SemaphoreType.DMA,
            pltpu.SemaphoreType.DMA,
        ],
        compiler_params=pltpu.CompilerParams(use_tc_tiling_on_sc=False,
                                             needs_layout_passes=False),
    )
    def _sc_gather(table_hbm, idx_hbm, out_hbm, table_v, idx_v, buf0, buf1,
                   tsem, wsem0, wsem1):
        b = lax.axis_index("s") * _NC + lax.axis_index("c")
        c1 = pltpu.async_copy(idx_hbm.at[pl.ds(b * tchunks, tchunks)], idx_v,
                              tsem)
        c2 = pltpu.async_copy(table_hbm, table_v, tsem)
        c1.wait()
        c2.wait()

        def assemble(tc, buf):
            for g in range(_LANES // 16):
                rows = idx_v[tc, pl.ds(g * 16, 16)]

                @plsc.parallel_loop(0, _EMBED_DIM, unroll=16)
                def _(e):
                    col = jnp.full((16,), 0, jnp.int32) + e
                    buf[e, pl.ds(g * 16, 16)] = plsc.load_gather(
                        table_v, [rows, col])

        def fire(tc, buf, wsem):
            for ts in range(_SUB):
                pltpu.async_copy(buf.at[pl.ds(ts * _SUB, _SUB)],
                                 out_hbm.at[b, ts, tc], wsem)

        def drain(tc, buf, wsem):
            for ts in range(_SUB):
                pltpu.make_async_copy(buf.at[pl.ds(ts * _SUB, _SUB)],
                                      out_hbm.at[b, ts, tc], wsem).wait()

        def body(i, carry):
            tc0 = 2 * i
            tc1 = tc0 + 1

            @pl.when(i > 0)
            def _():
                drain(tc0 - 2, buf0, wsem0)

            assemble(tc0, buf0)
            fire(tc0, buf0, wsem0)

            @pl.when(i > 0)
            def _():
                drain(tc1 - 2, buf1, wsem1)

            assemble(tc1, buf1)
            fire(tc1, buf1, wsem1)
            return carry

        lax.fori_loop(0, tchunks // 2, body, 0)
        drain(tchunks - 2, buf0, wsem0)
        drain(tchunks - 1, buf1, wsem1)

    return _sc_gather


def kernel(x, embeddings):
    xt = jnp.swapaxes(x, 1, 2)
    embt = embeddings.T
    idx, loss = _tc_stage(xt, embt)

    embt_pad = jnp.pad(embt, ((0, 0), (0, 1)))
    out5 = _make_sc_gather()(embt_pad, idx)
    qtised = out5.transpose(0, 2, 4, 1, 3).reshape(_B, _T, _EMBED_DIM)
    return (qtised, loss.reshape(()))
